# speedup vs baseline: 79.8187x; 79.8187x over previous
import functools

import jax
import jax.numpy as jnp
from jax import lax
from jax.experimental import pallas as pl

_T = 0.3
_L = 128


def _nms_body(x1r, y1r, x2r, y2r, keep_ref, *, nrow):
    ia = lax.broadcasted_iota(jnp.int32, (_L, _L), 0)
    ib = lax.broadcasted_iota(jnp.int32, (_L, _L), 1)
    eye = ia == ib

    def to_col(vrow):
        m = jnp.where(eye, jnp.broadcast_to(vrow, (_L, _L)), 0.0)
        return jnp.sum(m, axis=1, keepdims=True)

    def to_row(vcol):
        m = jnp.where(eye, jnp.broadcast_to(vcol, (_L, _L)), 0.0)
        return jnp.sum(m, axis=0, keepdims=True)

    keep_ref[...] = jnp.ones((nrow, _L), jnp.float32)

    def blk_body(blk, carry):
        rx1 = x1r[pl.ds(blk, 1), :]
        ry1 = y1r[pl.ds(blk, 1), :]
        rx2 = x2r[pl.ds(blk, 1), :]
        ry2 = y2r[pl.ds(blk, 1), :]
        qx1, qy1, qx2, qy2 = to_col(rx1), to_col(ry1), to_col(rx2), to_col(ry2)
        qarea = (qx2 - qx1) * (qy2 - qy1)
        rarea = (rx2 - rx1) * (ry2 - ry1)
        qkeep = to_col(keep_ref[pl.ds(blk, 1), :])

        xx1 = jnp.maximum(qx1, rx1)
        yy1 = jnp.maximum(qy1, ry1)
        xx2 = jnp.minimum(qx2, rx2)
        yy2 = jnp.minimum(qy2, ry2)
        inter = jnp.maximum(xx2 - xx1, 0.0) * jnp.maximum(yy2 - yy1, 0.0)
        iou = inter / (qarea + rarea - inter)
        pmat = (iou > _T) & (ib < ia)

        def fbody(c):
            k, _ = c
            kr = to_row(k)
            s = jnp.max(jnp.where(pmat, jnp.broadcast_to(kr, (_L, _L)), 0.0),
                        axis=1, keepdims=True)
            kn = qkeep * (1.0 - s)
            return kn, jnp.any(kn != k)

        k1 = fbody((qkeep, True))
        kfin, _ = lax.while_loop(lambda c: c[1], fbody, k1)

        def tail_body(r, c):
            tx1 = x1r[pl.ds(r, 1), :]
            ty1 = y1r[pl.ds(r, 1), :]
            tx2 = x2r[pl.ds(r, 1), :]
            ty2 = y2r[pl.ds(r, 1), :]
            tarea = (tx2 - tx1) * (ty2 - ty1)
            xx1t = jnp.maximum(qx1, tx1)
            yy1t = jnp.maximum(qy1, ty1)
            xx2t = jnp.minimum(qx2, tx2)
            yy2t = jnp.minimum(qy2, ty2)
            intr = jnp.maximum(xx2t - xx1t, 0.0) * jnp.maximum(yy2t - yy1t, 0.0)
            iou_t = intr / (qarea + tarea - intr)
            ok = (iou_t > _T) & ((r * _L + ib) > (blk * _L + ia))
            s = jnp.max(jnp.where(ok, jnp.broadcast_to(kfin, (_L, _L)), 0.0),
                        axis=0, keepdims=True)
            keep_ref[pl.ds(r, 1), :] = keep_ref[pl.ds(r, 1), :] * (1.0 - s)
            return c

        lax.fori_loop(blk, nrow, tail_body, 0)
        return carry

    lax.fori_loop(0, nrow, blk_body, 0)


@functools.partial(jax.jit, static_argnames=("nrow", "interpret"))
def _nms_sorted(x1, y1, x2, y2, *, nrow, interpret=False):
    body = functools.partial(_nms_body, nrow=nrow)
    return pl.pallas_call(
        body,
        out_shape=jax.ShapeDtypeStruct((nrow, _L), jnp.float32),
        interpret=interpret,
    )(x1, y1, x2, y2)


def _nms_keep_sorted(boxes_sorted, n, interpret=False):
    nrow = (n + _L - 1) // _L
    npad = nrow * _L
    bs = jnp.pad(boxes_sorted, ((0, npad - n), (0, 0)))
    x1 = bs[:, 0].reshape(nrow, _L)
    y1 = bs[:, 1].reshape(nrow, _L)
    x2 = bs[:, 2].reshape(nrow, _L)
    y2 = bs[:, 3].reshape(nrow, _L)
    keepf = _nms_sorted(x1, y1, x2, y2, nrow=nrow, interpret=interpret)
    return keepf.reshape(-1)[:n] > 0.0


def kernel(boxes, scores):
    n = scores.shape[0]
    order = jnp.argsort(-scores)
    keep_sorted = _nms_keep_sorted(jnp.take(boxes, order, axis=0), n)
    keep = jnp.zeros((n,), bool).at[order].set(keep_sorted)
    masked_scores = scores * keep.astype(scores.dtype)
    return masked_scores, keep.astype(jnp.int32)
